# baseline (speedup 1.0000x reference)
import jax
import jax.numpy as jnp
from jax import lax
from jax.experimental import pallas as pl
from jax.experimental.pallas import tpu as pltpu
from jax.experimental.pallas import tpu_sc as plsc

PRE_SEQ_LEN = 128
HIDDEN = 1024
NUM_LAYERS = 24
OUT_DIM = NUM_LAYERS * 2 * HIDDEN
BATCH = 4

NB = BATCH * PRE_SEQ_LEN
SPLIT = 32
DC = OUT_DIM // SPLIT
NBUF = 4

NC, NS, L = 2, 16, 16
NW = NC * NS
B_PER_W = NB // NW


def _body(table, sidx_hbm, perm_hbm, out, idx_v, perm_v, buf,
          gsem0, gsem1, gsem2, gsem3, wsem0, wsem1, wsem2, wsem3):
    wid = lax.axis_index("s") * NC + lax.axis_index("c")
    base = wid * B_PER_W
    gsems = (gsem0, gsem1, gsem2, gsem3)
    wsems = (wsem0, wsem1, wsem2, wsem3)

    pltpu.sync_copy(sidx_hbm.at[pl.ds(base, B_PER_W)], idx_v)
    pltpu.sync_copy(perm_hbm.at[pl.ds(base, B_PER_W)], perm_v)
    sv = idx_v[...]
    pv = perm_v[...]

    lane = lax.iota(jnp.int32, L)
    s = [jnp.sum(jnp.where(lane == j, sv, 0)) for j in range(B_PER_W)]
    p = [jnp.sum(jnp.where(lane == j, pv, 0)) for j in range(B_PER_W)]
    f = [None] + [s[j] != s[j - 1] for j in range(1, B_PER_W)]
    r = [jnp.int32(0)]
    for j in range(1, B_PER_W):
        r.append(r[j - 1] + jnp.where(f[j], 1, 0).astype(jnp.int32))

    def gsrc(c, j):
        return table.at[pl.ds(s[j], 1), pl.ds(c * DC, DC)]

    def brow(i, j):
        return buf.at[pl.ds(i * B_PER_W + r[j], 1)]

    def wdst(c, j):
        return out.at[pl.ds(p[j], 1), pl.ds(c * DC, DC)]

    def issue_gathers(c, i):
        pltpu.async_copy(gsrc(c, 0), brow(i, 0), gsems[i])
        for j in range(1, B_PER_W):
            @pl.when(f[j])
            def _(c=c, i=i, j=j):
                pltpu.async_copy(gsrc(c, j), brow(i, j), gsems[i])

    def drain_gathers(c, i):
        pltpu.make_async_copy(gsrc(c, 0), brow(i, 0), gsems[i]).wait()
        for j in range(1, B_PER_W):
            @pl.when(f[j])
            def _(c=c, i=i, j=j):
                pltpu.make_async_copy(gsrc(c, j), brow(i, j), gsems[i]).wait()

    def issue_writes(c, i):
        for j in range(B_PER_W):
            pltpu.async_copy(brow(i, j), wdst(c, j), wsems[i])

    def drain_writes_all(i):
        pltpu.make_async_copy(
            buf.at[pl.ds(i * B_PER_W, B_PER_W)],
            out.at[pl.ds(0, B_PER_W), pl.ds(0, DC)],
            wsems[i],
        ).wait()

    def q_body(q, carry):
        c0 = q * NBUF
        for i in range(NBUF):
            c = c0 + i

            @pl.when(q >= 1)
            def _(i=i):
                drain_writes_all(i)

            issue_gathers(c, i)

            def _tail(c=c, i=i):
                drain_gathers(c - 1, (i - 1) % NBUF)
                issue_writes(c - 1, (i - 1) % NBUF)

            if i == 0:
                pl.when(q >= 1)(_tail)
            else:
                _tail()
        return carry

    lax.fori_loop(0, SPLIT // NBUF, q_body, jnp.int32(0))

    last = SPLIT - 1
    drain_gathers(last, last % NBUF)
    issue_writes(last, last % NBUF)
    for i in range(NBUF):
        drain_writes_all(i)


@jax.jit
def _sc_gather(table, sidx, perm):
    mesh = plsc.VectorSubcoreMesh(core_axis_name="c", subcore_axis_name="s")
    k = pl.kernel(
        _body,
        out_type=jax.ShapeDtypeStruct((NB, OUT_DIM), jnp.float32),
        mesh=mesh,
        compiler_params=pltpu.CompilerParams(needs_layout_passes=False),
        scratch_types=(
            [pltpu.VMEM((B_PER_W,), jnp.int32)] * 2
            + [pltpu.VMEM((NBUF * B_PER_W, DC), jnp.float32)]
            + [pltpu.SemaphoreType.DMA] * (2 * NBUF)
        ),
    )
    return k(table, sidx, perm)


def kernel(prefix, embedding_weight):
    idx = prefix.reshape(NB)
    pos = lax.iota(jnp.int32, NB)
    sidx, perm = lax.sort_key_val(idx, pos)
    out = _sc_gather(embedding_weight, sidx, perm)
    return out.reshape(BATCH, PRE_SEQ_LEN, OUT_DIM)


# speedup vs baseline: 1.3338x; 1.3338x over previous
import jax
import jax.numpy as jnp
from jax import lax
from jax.experimental import pallas as pl
from jax.experimental.pallas import tpu as pltpu
from jax.experimental.pallas import tpu_sc as plsc

PRE_SEQ_LEN = 128
HIDDEN = 1024
NUM_LAYERS = 24
OUT_DIM = NUM_LAYERS * 2 * HIDDEN
BATCH = 4

NB = BATCH * PRE_SEQ_LEN
SPLIT = 16
DC = OUT_DIM // SPLIT
NBUF = 2

NC, NS, L = 2, 16, 16
NW = NC * NS
B_PER_W = NB // NW


def _body(table, sidx_hbm, perm_hbm, out, idx_v, perm_v, buf, *sems):
    wid = lax.axis_index("s") * NC + lax.axis_index("c")
    base = wid * B_PER_W
    gsems = sems[:NBUF]
    wsems = sems[NBUF:]

    pltpu.sync_copy(sidx_hbm.at[pl.ds(base, B_PER_W)], idx_v)
    pltpu.sync_copy(perm_hbm.at[pl.ds(base, B_PER_W)], perm_v)
    sv = idx_v[...]
    pv = perm_v[...]

    lane = lax.iota(jnp.int32, L)
    s = [jnp.sum(jnp.where(lane == j, sv, 0)) for j in range(B_PER_W)]
    p = [jnp.sum(jnp.where(lane == j, pv, 0)) for j in range(B_PER_W)]
    f = [None] + [s[j] != s[j - 1] for j in range(1, B_PER_W)]
    r = [jnp.int32(0)]
    for j in range(1, B_PER_W):
        r.append(r[j - 1] + jnp.where(f[j], 1, 0).astype(jnp.int32))

    def gsrc(c, j):
        return table.at[pl.ds(s[j], 1), pl.ds(c * DC, DC)]

    def brow(i, j):
        return buf.at[pl.ds(i * B_PER_W + r[j], 1)]

    def wdst(c, j):
        return out.at[pl.ds(p[j], 1), pl.ds(c * DC, DC)]

    def issue_gathers(c, i):
        pltpu.async_copy(gsrc(c, 0), brow(i, 0), gsems[i])
        for j in range(1, B_PER_W):
            @pl.when(f[j])
            def _(c=c, i=i, j=j):
                pltpu.async_copy(gsrc(c, j), brow(i, j), gsems[i])

    def drain_gathers(c, i):
        pltpu.make_async_copy(gsrc(c, 0), brow(i, 0), gsems[i]).wait()
        for j in range(1, B_PER_W):
            @pl.when(f[j])
            def _(c=c, i=i, j=j):
                pltpu.make_async_copy(gsrc(c, j), brow(i, j), gsems[i]).wait()

    def issue_writes(c, i):
        for j in range(B_PER_W):
            pltpu.async_copy(brow(i, j), wdst(c, j), wsems[i])

    def drain_writes_all(i):
        pltpu.make_async_copy(
            buf.at[pl.ds(i * B_PER_W, B_PER_W)],
            out.at[pl.ds(0, B_PER_W), pl.ds(0, DC)],
            wsems[i],
        ).wait()

    def q_body(q, carry):
        c0 = q * NBUF
        for i in range(NBUF):
            c = c0 + i

            @pl.when(q >= 1)
            def _(i=i):
                drain_writes_all(i)

            issue_gathers(c, i)

            def _tail(c=c, i=i):
                drain_gathers(c - 1, (i - 1) % NBUF)
                issue_writes(c - 1, (i - 1) % NBUF)

            if i == 0:
                pl.when(q >= 1)(_tail)
            else:
                _tail()
        return carry

    lax.fori_loop(0, SPLIT // NBUF, q_body, jnp.int32(0))

    last = SPLIT - 1
    drain_gathers(last, last % NBUF)
    issue_writes(last, last % NBUF)
    for i in range(NBUF):
        drain_writes_all(i)


@jax.jit
def _sc_gather(table, sidx, perm):
    mesh = plsc.VectorSubcoreMesh(core_axis_name="c", subcore_axis_name="s")
    k = pl.kernel(
        _body,
        out_type=jax.ShapeDtypeStruct((NB, OUT_DIM), jnp.float32),
        mesh=mesh,
        compiler_params=pltpu.CompilerParams(needs_layout_passes=False),
        scratch_types=(
            [pltpu.VMEM((B_PER_W,), jnp.int32)] * 2
            + [pltpu.VMEM((NBUF * B_PER_W, DC), jnp.float32)]
            + [pltpu.SemaphoreType.DMA] * (2 * NBUF)
        ),
    )
    return k(table, sidx, perm)


def kernel(prefix, embedding_weight):
    idx = prefix.reshape(NB)
    pos = lax.iota(jnp.int32, NB)
    sidx, perm = lax.sort_key_val(idx, pos)
    out = _sc_gather(embedding_weight, sidx, perm)
    return out.reshape(BATCH, PRE_SEQ_LEN, OUT_DIM)
